# baseline (speedup 1.0000x reference)
import jax, jax.numpy as jnp

N = 10000
NREL = 1000


def _seg_softmax(logits, seg, num_segments):
    m = jax.ops.segment_max(logits, seg, num_segments=num_segments)
    ex = jnp.exp(logits - m[seg])
    s = jax.ops.segment_sum(ex, seg, num_segments=num_segments)
    return ex / s[seg]


def _gcn(x, ei):
    src, dst = ei[0], ei[1]
    n = x.shape[0]
    deg = jax.ops.segment_sum(jnp.ones((ei.shape[1],), dtype=x.dtype), dst, num_segments=n)
    dis = deg ** -0.5
    norm = dis[src] * dis[dst]
    out = jax.ops.segment_sum(norm[:, None] * x[src], dst, num_segments=n)
    return jax.nn.relu(out)


def _highway(x1, x2, W, b):
    gate = jax.nn.sigmoid(x1 @ W + b)
    return gate * x2 + (1.0 - gate) * x1


def kernel(x_e, edge_index, rel, edge_index_all, rel_all, hw1_W, hw1_b, hw2_W, hw2_b,
           e2r_ah1, e2r_ah2, e2r_at1, e2r_at2, e2r_wh, e2r_wt,
           r2e_ah, r2e_at, r2e_ar, gat_ai, gat_aj):
    n = x_e.shape[0]
    n_rel = NREL
    x = _highway(x_e, _gcn(x_e, edge_index_all), hw1_W, hw1_b)
    x = _highway(x, _gcn(x, edge_index_all), hw2_W, hw2_b)
    h, t = edge_index[0], edge_index[1]
    x_r_h = x @ e2r_wh
    x_r_t = x @ e2r_wt
    e1 = (x_r_h @ e2r_ah1)[h] + (x_r_t @ e2r_ah2)[t]
    e2 = (x_r_h @ e2r_at1)[h] + (x_r_t @ e2r_at2)[t]
    a1 = _seg_softmax(jax.nn.leaky_relu(e1), rel, n_rel)
    xrh = jax.ops.segment_sum(a1[:, None] * x_r_h[h], rel, num_segments=n_rel)
    a2 = _seg_softmax(jax.nn.leaky_relu(e2), rel, n_rel)
    xrt = jax.ops.segment_sum(a2[:, None] * x_r_t[t], rel, num_segments=n_rel)
    x_r = xrh + xrt
    e_h = (x @ r2e_ah)[h]
    e_t = (x @ r2e_at)[t]
    e_r = (x_r @ r2e_ar)[rel]
    ah = _seg_softmax(jax.nn.leaky_relu(e_h + e_r), h, n)
    x_e_h = jax.ops.segment_sum(ah[:, None] * x_r[rel], h, num_segments=n)
    at = _seg_softmax(jax.nn.leaky_relu(e_t + e_r), t, n)
    x_e_t = jax.ops.segment_sum(at[:, None] * x_r[rel], t, num_segments=n)
    x = jnp.concatenate([x, x_e_h, x_e_t], axis=1)
    src, dst = edge_index_all[0], edge_index_all[1]
    e_i = (x @ gat_ai)[dst]
    e_j = (x @ gat_aj)[src]
    ag = _seg_softmax(jax.nn.leaky_relu(e_i + e_j), dst, n)
    x_g = jax.nn.relu(jax.ops.segment_sum(ag[:, None] * x[src], dst, num_segments=n))
    return jnp.concatenate([x, x_g], axis=1)


# speedup vs baseline: 26.6528x; 26.6528x over previous
import functools

import jax
import jax.numpy as jnp
from jax import lax
from jax.experimental import pallas as pl
from jax.experimental.pallas import tpu as pltpu
from jax.experimental.pallas import tpu_sc as plsc

N = 10000
E = 640000
EH = 128
RH = 32
NREL = 1000

NC = 2
NS = 16
NW = NC * NS
L = 16

K = 128
NJ = 16
CHUNK = NJ * K
NCHUNK = 10
E_PER = NCHUNK * CHUNK
E_PAD = NW * E_PER

NP = 10240
NRELP = 1024

_MESH = dict(core_axis_name="c", subcore_axis_name="s", num_cores=NC,
             num_subcores=NS)


def _sc_edge_pass(mode, d, n_out, np_out, n_u, n_v, has_feat, has_sacc,
                  w_out, n_in, uv_same=False, i1_is_gidx=False):
    out_type = []
    if has_feat:
        out_type.append(jax.ShapeDtypeStruct((NC * np_out, d), jnp.float32))
    if has_sacc:
        out_type.append(jax.ShapeDtypeStruct((NC * np_out, L), jnp.float32))
    if w_out:
        out_type.append(jax.ShapeDtypeStruct((E_PAD,), jnp.float32))

    scratch = {}
    if has_feat:
        scratch["facc"] = pltpu.VMEM_SHARED((np_out, d), jnp.float32)
        scratch["gidx_v"] = pltpu.VMEM((CHUNK,), jnp.int32)
        scratch["rows_v"] = pltpu.VMEM((K, d), jnp.float32)
    if has_sacc:
        scratch["sacc"] = pltpu.VMEM_SHARED((np_out, L), jnp.float32)
        scratch["brows_v"] = pltpu.VMEM((K, L), jnp.float32)
    scratch["sidx_v"] = pltpu.VMEM((NJ, K), jnp.int32)
    if mode in ("gat", "gcn"):
        if not i1_is_gidx:
            scratch["i1_v"] = pltpu.VMEM((CHUNK,), jnp.int32)
        scratch["i2_v"] = pltpu.VMEM((CHUNK,), jnp.int32)
        scratch["u_v"] = pltpu.VMEM((n_u,), jnp.float32)
        if not uv_same:
            scratch["v_v"] = pltpu.VMEM((n_v,), jnp.float32)
    if mode != "ones":
        scratch["w_v"] = pltpu.VMEM((CHUNK,), jnp.float32)
    scratch["sem"] = pltpu.SemaphoreType.DMA
    snames = list(scratch.keys())

    def body(*refs):
        nin = 0
        args = {}
        if has_feat:
            args["x_hbm"] = refs[nin]; nin += 1
            args["gidx_hbm"] = refs[nin]; nin += 1
        args["sidx_hbm"] = refs[nin]; nin += 1
        if mode in ("gat", "gcn"):
            if not i1_is_gidx:
                args["i1_hbm"] = refs[nin]; nin += 1
            args["i2_hbm"] = refs[nin]; nin += 1
            args["u_hbm"] = refs[nin]; nin += 1
            if not uv_same:
                args["v_hbm"] = refs[nin]; nin += 1
        if mode == "load":
            args["wsrc_hbm"] = refs[nin]; nin += 1
        if has_feat:
            args["facc_out"] = refs[nin]; nin += 1
        if has_sacc:
            args["sacc_out"] = refs[nin]; nin += 1
        if w_out:
            args["w_hbm"] = refs[nin]; nin += 1
        for nm, r in zip(snames, refs[nin:]):
            args[nm] = r
        if mode in ("gat", "gcn"):
            if i1_is_gidx:
                args["i1_v"] = args["gidx_v"]
            if uv_same:
                args["v_v"] = args["u_v"]

        c = lax.axis_index("c")
        s = lax.axis_index("s")
        wid = c * NS + s

        if mode in ("gat", "gcn"):
            pltpu.sync_copy(args["u_hbm"], args["u_v"])
            if not uv_same:
                pltpu.sync_copy(args["v_hbm"], args["v_v"])

        zero16 = jnp.zeros((L,), jnp.float32)
        rows_per = np_out // NS
        base_rows = s * rows_per

        def zero_buf(zref, width):
            def zrow(i, _):
                for dd in range(width // L):
                    zref[i, pl.ds(dd * L, L)] = zero16
                return 0
            lax.fori_loop(0, K, zrow, 0)

        def zero_acc(zref, acc):
            zr = min(rows_per, K)
            def zstep(r, _):
                pltpu.sync_copy(zref.at[pl.ds(0, zr)],
                                acc.at[pl.ds(base_rows + r * zr, zr)])
                return 0
            lax.fori_loop(0, rows_per // zr, zstep, 0)

        if has_feat:
            zero_buf(args["rows_v"], d)
            zero_acc(args["rows_v"], args["facc"])
        if has_sacc:
            zero_buf(args["brows_v"], L)
            zero_acc(args["brows_v"], args["sacc"])
        plsc.subcore_barrier()

        if mode == "ones":
            one16 = jnp.full((L,), 1.0, jnp.float32)

            def orow(i, _):
                args["brows_v"][i, :] = one16
                return 0
            lax.fori_loop(0, K, orow, 0)

        def chunk_body(ci, _):
            ebase = (wid * NCHUNK + ci) * CHUNK
            rbase = (wid * NCHUNK + ci) * NJ
            pltpu.sync_copy(args["sidx_hbm"].at[pl.ds(rbase, NJ)],
                            args["sidx_v"])
            if has_feat:
                pltpu.sync_copy(args["gidx_hbm"].at[pl.ds(ebase, CHUNK)],
                                args["gidx_v"])
            if mode in ("gat", "gcn"):
                if not i1_is_gidx:
                    pltpu.sync_copy(args["i1_hbm"].at[pl.ds(ebase, CHUNK)],
                                    args["i1_v"])
                pltpu.sync_copy(args["i2_hbm"].at[pl.ds(ebase, CHUNK)],
                                args["i2_v"])

                def wstep(i, _):
                    off = i * L
                    idx1 = args["i1_v"][pl.ds(off, L)]
                    idx2 = args["i2_v"][pl.ds(off, L)]
                    a = plsc.load_gather(args["u_v"], [idx1])
                    b = plsc.load_gather(args["v_v"], [idx2])
                    if mode == "gat":
                        z = a + b
                        w16 = jnp.exp(jnp.maximum(z, 0.01 * z))
                    else:
                        w16 = a * b
                    args["w_v"][pl.ds(off, L)] = w16
                    return 0
                lax.fori_loop(0, CHUNK // L, wstep, 0)
            if mode == "load":
                pltpu.sync_copy(args["wsrc_hbm"].at[pl.ds(ebase, CHUNK)],
                                args["w_v"])
            if w_out:
                pltpu.sync_copy(args["w_v"],
                                args["w_hbm"].at[pl.ds(ebase, CHUNK)])

            def jstep(j, _):
                if has_feat:
                    pltpu.async_copy(
                        args["x_hbm"].at[args["gidx_v"].at[pl.ds(j * K, K)]],
                        args["rows_v"], args["sem"]).wait()
                if mode != "ones":
                    def scale_grp(ii, _):
                        rbase = ii * L
                        w16 = args["w_v"][pl.ds(j * K + rbase, L)]
                        for l in range(L):
                            wb = lax.broadcast(w16[l], (L,))
                            row = rbase + l
                            if has_feat:
                                for dd in range(d // L):
                                    sl = pl.ds(dd * L, L)
                                    args["rows_v"][row, sl] = (
                                        args["rows_v"][row, sl] * wb)
                            if has_sacc:
                                args["brows_v"][row, :] = wb
                        return 0
                    lax.fori_loop(0, K // L, scale_grp, 0)
                row_idx = args["sidx_v"].at[j]
                if has_feat:
                    pltpu.sync_copy(args["rows_v"],
                                    args["facc"].at[row_idx], add=True)
                if has_sacc:
                    pltpu.sync_copy(args["brows_v"],
                                    args["sacc"].at[row_idx], add=True)
                return 0
            lax.fori_loop(0, NJ, jstep, 0)
            return 0
        lax.fori_loop(0, NCHUNK, chunk_body, 0)

        plsc.subcore_barrier()
        obase = c * np_out + base_rows
        if has_feat:
            pltpu.sync_copy(args["facc"].at[pl.ds(base_rows, rows_per)],
                            args["facc_out"].at[pl.ds(obase, rows_per)])
        if has_sacc:
            pltpu.sync_copy(args["sacc"].at[pl.ds(base_rows, rows_per)],
                            args["sacc_out"].at[pl.ds(obase, rows_per)])

    mesh = plsc.VectorSubcoreMesh(**_MESH)
    return pl.kernel(body, out_type=tuple(out_type), mesh=mesh,
                     scratch_types=list(scratch.values()),
                     compiler_params=pltpu.CompilerParams(
                         needs_layout_passes=False,
                         use_tc_tiling_on_sc=False))


R = 2048
GN = 5


def _tc_call(body, out_type):
    return pl.pallas_call(body, out_shape=out_type)


def _rows(w):
    return pl.BlockSpec((R, w), lambda i: (i, 0))


def _acc3(w):
    return pl.BlockSpec((NC, R, w), lambda i: (0, i, 0))


def _full(*shape):
    nd = len(shape)
    return pl.BlockSpec(shape, lambda i: (0,) * nd)


def _vec():
    return pl.BlockSpec((R,), lambda i: (i,))


def _inv0(s3):
    s0 = s3[0, :, 0] + s3[1, :, 0]
    return jnp.where(s0 > 0, 1.0 / s0, 0.0)[:, None]


def _dis_body(sacc_ref, out_ref):
    a = sacc_ref[...]
    deg = a[0, :, 0] + a[1, :, 0]
    out_ref[...] = jnp.where(deg > 0, lax.rsqrt(jnp.maximum(deg, 1e-30)), 0.0)


def _hw_body(xin_ref, gp_ref, w_ref, b_ref, out_ref):
    gp = gp_ref[...]
    g = jax.nn.relu(gp[0] + gp[1])
    xin = xin_ref[...]
    gate = jax.nn.sigmoid(
        jnp.dot(xin, w_ref[...], preferred_element_type=jnp.float32)
        + b_ref[...])
    out_ref[...] = gate * g + (1.0 - gate) * xin


def _proj_body(x_ref, wh_ref, wt_ref, ah1_ref, ah2_ref, at1_ref, at2_ref,
               rah_ref, rat_ref,
               xrh_ref, xrt_ref, ph1_ref, ph2_ref, pt1_ref, pt2_ref,
               ehn_ref, etn_ref):
    x = x_ref[...]
    xrh = jnp.dot(x, wh_ref[...], preferred_element_type=jnp.float32)
    xrt = jnp.dot(x, wt_ref[...], preferred_element_type=jnp.float32)
    xrh_ref[...] = xrh
    xrt_ref[...] = xrt
    ph1_ref[...] = jnp.sum(xrh * ah1_ref[...], axis=1)
    ph2_ref[...] = jnp.sum(xrt * ah2_ref[...], axis=1)
    pt1_ref[...] = jnp.sum(xrh * at1_ref[...], axis=1)
    pt2_ref[...] = jnp.sum(xrt * at2_ref[...], axis=1)
    ehn_ref[...] = jnp.sum(x * rah_ref[...], axis=1)
    etn_ref[...] = jnp.sum(x * rat_ref[...], axis=1)


def _xr_body(fh_ref, sh_ref, ft_ref, st_ref, ar_ref, xr_ref, rp_ref):
    fh = fh_ref[...]
    ft = ft_ref[...]
    sh = sh_ref[...]
    st = st_ref[...]
    sh0 = sh[0, :NREL, 0] + sh[1, :NREL, 0]
    st0 = st[0, :NREL, 0] + st[1, :NREL, 0]
    inv_h = jnp.where(sh0 > 0, 1.0 / sh0, 0.0)[:, None]
    inv_t = jnp.where(st0 > 0, 1.0 / st0, 0.0)[:, None]
    xr = (fh[0, :NREL, :] + fh[1, :NREL, :]) * inv_h \
        + (ft[0, :NREL, :] + ft[1, :NREL, :]) * inv_t
    xr_ref[...] = xr
    rp_ref[...] = jnp.sum(xr * ar_ref[...], axis=1)


def _cat_body(x_ref, fh_ref, sh_ref, ft_ref, st_ref, ai_ref, aj_ref,
              xcat_ref, gi_ref, gj_ref):
    fh = fh_ref[...]
    ft = ft_ref[...]
    xeh = (fh[0] + fh[1]) * _inv0(sh_ref[...])
    xet = (ft[0] + ft[1]) * _inv0(st_ref[...])
    xcat = jnp.concatenate([x_ref[...], xeh, xet], axis=1)
    xcat_ref[...] = xcat
    gi_ref[...] = jnp.sum(xcat * ai_ref[...], axis=1)
    gj_ref[...] = jnp.sum(xcat * aj_ref[...], axis=1)


def _out_body(xcat_ref, fg_ref, sg_ref, out_ref):
    fg = fg_ref[...]
    xg = jax.nn.relu((fg[0] + fg[1]) * _inv0(sg_ref[...]))
    out_ref[...] = jnp.concatenate([xcat_ref[...], xg], axis=1)


def _padi(a, fill):
    return jnp.concatenate(
        [a, jnp.full((E_PAD - E,), fill, a.dtype)])


@jax.jit
def _run(x_e, edge_index, rel, edge_index_all,
         hw1_W, hw1_b, hw2_W, hw2_b,
         e2r_ah1, e2r_ah2, e2r_at1, e2r_at2, e2r_wh, e2r_wt,
         r2e_ah, r2e_at, r2e_ar, gat_ai, gat_aj):
    f32 = jnp.float32
    src_a = edge_index_all[0]
    dst_a = edge_index_all[1]
    h = edge_index[0]
    t = edge_index[1]

    src_a_g = _padi(src_a, 0)
    dst_a_g = _padi(dst_a, 0)
    dst_a_s = _padi(dst_a, N).reshape(E_PAD // K, K)
    h_g = _padi(h, 0)
    t_g = _padi(t, 0)
    rel_g = _padi(rel, 0)
    h_s = _padi(h, N).reshape(E_PAD // K, K)
    t_s = _padi(t, N).reshape(E_PAD // K, K)
    rel_s = _padi(rel, NREL).reshape(E_PAD // K, K)

    deg_pass = _sc_edge_pass("ones", 0, N, NP, 0, 0, False, True, False, 0)
    (sacc_deg,) = deg_pass(dst_a_s)
    dis = pl.pallas_call(
        _dis_body, grid=(GN,), in_specs=[_acc3(L)], out_specs=_vec(),
        out_shape=jax.ShapeDtypeStruct((N,), f32))(
        sacc_deg.reshape(NC, NP, L))

    gcn = _sc_edge_pass("gcn", EH, N, NP, N, N, True, False, False, N,
                        uv_same=True, i1_is_gidx=True)
    (g1,) = gcn(x_e, src_a_g, dst_a_s, dst_a_g, dis)
    hw_call = pl.pallas_call(
        _hw_body, grid=(GN,),
        in_specs=[_rows(EH), _acc3(EH), _full(EH, EH), _full(1, EH)],
        out_specs=_rows(EH),
        out_shape=jax.ShapeDtypeStruct((N, EH), f32))
    x1 = hw_call(x_e, g1.reshape(NC, NP, EH), hw1_W, hw1_b.reshape(1, EH))

    (g2,) = gcn(x1, src_a_g, dst_a_s, dst_a_g, dis)
    x = hw_call(x1, g2.reshape(NC, NP, EH), hw2_W, hw2_b.reshape(1, EH))

    outs = pl.pallas_call(
        _proj_body, grid=(GN,),
        in_specs=[_rows(EH), _full(EH, RH), _full(EH, RH)]
        + [_full(1, RH)] * 4 + [_full(1, EH)] * 2,
        out_specs=(_rows(RH), _rows(RH)) + (_vec(),) * 6,
        out_shape=(
            jax.ShapeDtypeStruct((N, RH), f32),
            jax.ShapeDtypeStruct((N, RH), f32),
        ) + (jax.ShapeDtypeStruct((N,), f32),) * 6,
    )(x, e2r_wh, e2r_wt,
      e2r_ah1.reshape(1, RH), e2r_ah2.reshape(1, RH),
      e2r_at1.reshape(1, RH), e2r_at2.reshape(1, RH),
      r2e_ah.reshape(1, EH), r2e_at.reshape(1, EH))
    xrh, xrt, ph1, ph2, pt1, pt2, ehn, etn = outs

    e2r = _sc_edge_pass("gat", RH, NREL, NRELP, N, N, True, True, False, N)
    fh, sh = e2r(xrh, h_g, rel_s, h_g, t_g, ph1, ph2)
    ft, st = e2r(xrt, t_g, rel_s, h_g, t_g, pt1, pt2)
    x_r, r_proj = _tc_call(_xr_body, (
        jax.ShapeDtypeStruct((NREL, RH), f32),
        jax.ShapeDtypeStruct((NREL,), f32),
    ))(fh.reshape(NC, NRELP, RH), sh.reshape(NC, NRELP, L),
       ft.reshape(NC, NRELP, RH), st.reshape(NC, NRELP, L),
       r2e_ar.reshape(1, RH))

    r2e = _sc_edge_pass("gat", RH, N, NP, N, NREL, True, True, False, NREL)
    fxh, sxh = r2e(x_r, rel_g, h_s, h_g, rel_g, ehn, r_proj)
    fxt, sxt = r2e(x_r, rel_g, t_s, t_g, rel_g, etn, r_proj)
    dcat = EH + 2 * RH
    xcat, gi, gj = pl.pallas_call(
        _cat_body, grid=(GN,),
        in_specs=[_rows(EH), _acc3(RH), _acc3(L), _acc3(RH), _acc3(L),
                  _full(1, dcat), _full(1, dcat)],
        out_specs=(_rows(dcat), _vec(), _vec()),
        out_shape=(
            jax.ShapeDtypeStruct((N, dcat), f32),
            jax.ShapeDtypeStruct((N,), f32),
            jax.ShapeDtypeStruct((N,), f32),
        ),
    )(x, fxh.reshape(NC, NP, RH), sxh.reshape(NC, NP, L),
      fxt.reshape(NC, NP, RH), sxt.reshape(NC, NP, L),
      gat_ai.reshape(1, dcat), gat_aj.reshape(1, dcat))

    fin_a = _sc_edge_pass("gat", 0, N, NP, N, N, False, True, True, 0)
    sg, w_all = fin_a(dst_a_s, dst_a_g, src_a_g, gi, gj)
    dh = dcat // 2
    fin_b = _sc_edge_pass("load", dh, N, NP, 0, 0, True, False, False, N)
    (fg0,) = fin_b(xcat[:, :dh], src_a_g, dst_a_s, w_all)
    (fg1,) = fin_b(xcat[:, dh:], src_a_g, dst_a_s, w_all)
    fg = jnp.concatenate([fg0.reshape(NC, NP, dh), fg1.reshape(NC, NP, dh)],
                         axis=2)

    return pl.pallas_call(
        _out_body, grid=(GN,),
        in_specs=[_rows(dcat), _acc3(dcat), _acc3(L)],
        out_specs=_rows(2 * dcat),
        out_shape=jax.ShapeDtypeStruct((N, 2 * dcat), f32))(
        xcat, fg, sg.reshape(NC, NP, L))


def kernel(x_e, edge_index, rel, edge_index_all, rel_all, hw1_W, hw1_b,
           hw2_W, hw2_b, e2r_ah1, e2r_ah2, e2r_at1, e2r_at2, e2r_wh,
           e2r_wt, r2e_ah, r2e_at, r2e_ar, gat_ai, gat_aj):
    return _run(x_e, edge_index, rel, edge_index_all,
                hw1_W, hw1_b, hw2_W, hw2_b,
                e2r_ah1, e2r_ah2, e2r_at1, e2r_at2, e2r_wh, e2r_wt,
                r2e_ah, r2e_at, r2e_ar, gat_ai, gat_aj)


# speedup vs baseline: 32.8561x; 1.2327x over previous
import functools

import jax
import jax.numpy as jnp
from jax import lax
from jax.experimental import pallas as pl
from jax.experimental.pallas import tpu as pltpu
from jax.experimental.pallas import tpu_sc as plsc

N = 10000
E = 640000
EH = 128
RH = 32
NREL = 1000

NC = 2
NS = 16
NW = NC * NS
L = 16

K = 128
NJ = 10
CHUNK = NJ * K
NCHUNK = 16
E_PER = NCHUNK * CHUNK
E_PAD = NW * E_PER

NP = 10240
NRELP = 1024

_MESH = dict(core_axis_name="c", subcore_axis_name="s", num_cores=NC,
             num_subcores=NS)


def _sc_edge_pass(mode, d, n_out, np_out, n_u, n_v, has_feat, has_sacc,
                  w_out, n_in, uv_same=False, i1_is_gidx=False):
    out_type = []
    if has_feat:
        out_type.append(jax.ShapeDtypeStruct((NC * np_out, d), jnp.float32))
    if has_sacc:
        out_type.append(jax.ShapeDtypeStruct((NC * np_out, L), jnp.float32))
    if w_out:
        out_type.append(jax.ShapeDtypeStruct((E_PAD,), jnp.float32))

    scratch = {}
    if has_feat:
        scratch["facc"] = pltpu.VMEM_SHARED((np_out, d), jnp.float32)
        scratch["gidx_v"] = pltpu.VMEM((CHUNK,), jnp.int32)
        scratch["rows_v"] = pltpu.VMEM((2, K, d), jnp.float32)
    if has_sacc:
        scratch["sacc"] = pltpu.VMEM_SHARED((np_out, L), jnp.float32)
        scratch["brows_v"] = pltpu.VMEM((K, L), jnp.float32)
    scratch["sidx_v"] = pltpu.VMEM((NJ, K), jnp.int32)
    if mode in ("gat", "gcn"):
        if not i1_is_gidx:
            scratch["i1_v"] = pltpu.VMEM((CHUNK,), jnp.int32)
        scratch["i2_v"] = pltpu.VMEM((CHUNK,), jnp.int32)
        scratch["u_v"] = pltpu.VMEM((n_u,), jnp.float32)
        if not uv_same:
            scratch["v_v"] = pltpu.VMEM((n_v,), jnp.float32)
    if mode != "ones":
        scratch["w_v"] = pltpu.VMEM((CHUNK,), jnp.float32)
    scratch["sem"] = pltpu.SemaphoreType.DMA
    if has_feat:
        scratch["sem2"] = pltpu.SemaphoreType.DMA
    snames = list(scratch.keys())

    def body(*refs):
        nin = 0
        args = {}
        if has_feat:
            args["x_hbm"] = refs[nin]; nin += 1
            args["gidx_hbm"] = refs[nin]; nin += 1
        args["sidx_hbm"] = refs[nin]; nin += 1
        if mode in ("gat", "gcn"):
            if not i1_is_gidx:
                args["i1_hbm"] = refs[nin]; nin += 1
            args["i2_hbm"] = refs[nin]; nin += 1
            args["u_hbm"] = refs[nin]; nin += 1
            if not uv_same:
                args["v_hbm"] = refs[nin]; nin += 1
        if mode == "load":
            args["wsrc_hbm"] = refs[nin]; nin += 1
        if has_feat:
            args["facc_out"] = refs[nin]; nin += 1
        if has_sacc:
            args["sacc_out"] = refs[nin]; nin += 1
        if w_out:
            args["w_hbm"] = refs[nin]; nin += 1
        for nm, r in zip(snames, refs[nin:]):
            args[nm] = r
        if mode in ("gat", "gcn"):
            if i1_is_gidx:
                args["i1_v"] = args["gidx_v"]
            if uv_same:
                args["v_v"] = args["u_v"]

        c = lax.axis_index("c")
        s = lax.axis_index("s")
        wid = c * NS + s

        if mode in ("gat", "gcn"):
            pltpu.sync_copy(args["u_hbm"], args["u_v"])
            if not uv_same:
                pltpu.sync_copy(args["v_hbm"], args["v_v"])

        zero16 = jnp.zeros((L,), jnp.float32)
        rows_per = np_out // NS
        base_rows = s * rows_per

        def zero_buf(zref, width):
            def zrow(i, _):
                for dd in range(width // L):
                    zref[i, pl.ds(dd * L, L)] = zero16
                return 0
            lax.fori_loop(0, K, zrow, 0)

        def zero_acc(zref, acc):
            zr = min(rows_per, K)
            def zstep(r, _):
                pltpu.sync_copy(zref.at[pl.ds(0, zr)],
                                acc.at[pl.ds(base_rows + r * zr, zr)])
                return 0
            lax.fori_loop(0, rows_per // zr, zstep, 0)

        if has_feat:
            zero_buf(args["rows_v"].at[0], d)
            zero_acc(args["rows_v"].at[0], args["facc"])
        if has_sacc:
            zero_buf(args["brows_v"], L)
            zero_acc(args["brows_v"], args["sacc"])
        plsc.subcore_barrier()

        if mode == "ones":
            one16 = jnp.full((L,), 1.0, jnp.float32)

            def orow(i, _):
                args["brows_v"][i, :] = one16
                return 0
            lax.fori_loop(0, K, orow, 0)

        sems = [args.get("sem"), args.get("sem2")]

        def chunk_body(ci, _):
            ebase = (wid * NCHUNK + ci) * CHUNK
            rbase = (wid * NCHUNK + ci) * NJ
            if has_feat:
                pltpu.sync_copy(args["gidx_hbm"].at[pl.ds(ebase, CHUNK)],
                                args["gidx_v"])

            def gather(j, b):
                return pltpu.async_copy(
                    args["x_hbm"].at[args["gidx_v"].at[pl.ds(j * K, K)]],
                    args["rows_v"].at[b], sems[b])

            cps = {}
            if has_feat:
                cps[0] = gather(0, 0)
            pltpu.sync_copy(args["sidx_hbm"].at[pl.ds(rbase, NJ)],
                            args["sidx_v"])
            if mode in ("gat", "gcn"):
                if not i1_is_gidx:
                    pltpu.sync_copy(args["i1_hbm"].at[pl.ds(ebase, CHUNK)],
                                    args["i1_v"])
                pltpu.sync_copy(args["i2_hbm"].at[pl.ds(ebase, CHUNK)],
                                args["i2_v"])

                def wstep(i, _):
                    off = i * L
                    idx1 = args["i1_v"][pl.ds(off, L)]
                    idx2 = args["i2_v"][pl.ds(off, L)]
                    a = plsc.load_gather(args["u_v"], [idx1])
                    b = plsc.load_gather(args["v_v"], [idx2])
                    if mode == "gat":
                        z = a + b
                        w16 = jnp.exp(jnp.maximum(z, 0.01 * z))
                    else:
                        w16 = a * b
                    args["w_v"][pl.ds(off, L)] = w16
                    return 0
                lax.fori_loop(0, CHUNK // L, wstep, 0)
            if mode == "load":
                pltpu.sync_copy(args["wsrc_hbm"].at[pl.ds(ebase, CHUNK)],
                                args["w_v"])
            if w_out:
                pltpu.sync_copy(args["w_v"],
                                args["w_hbm"].at[pl.ds(ebase, CHUNK)])

            for j in range(NJ):
                b = j & 1
                if has_feat:
                    cps[b].wait()
                    if j + 1 < NJ:
                        cps[1 - b] = gather(j + 1, 1 - b)
                    rbuf = args["rows_v"].at[b]
                if mode != "ones":
                    def scale_grp(ii, _):
                        gbase = ii * L
                        w16 = args["w_v"][pl.ds(j * K + gbase, L)]
                        for l in range(L):
                            wb = lax.broadcast(w16[l], (L,))
                            row = gbase + l
                            if has_feat:
                                for dd in range(d // L):
                                    sl = pl.ds(dd * L, L)
                                    rbuf[row, sl] = rbuf[row, sl] * wb
                            if has_sacc:
                                args["brows_v"][row, :] = wb
                        return 0
                    lax.fori_loop(0, K // L, scale_grp, 0)
                row_idx = args["sidx_v"].at[j]
                if has_feat:
                    pltpu.sync_copy(rbuf, args["facc"].at[row_idx],
                                    add=True)
                if has_sacc:
                    pltpu.sync_copy(args["brows_v"],
                                    args["sacc"].at[row_idx], add=True)
            return 0
        lax.fori_loop(0, NCHUNK, chunk_body, 0)

        plsc.subcore_barrier()
        obase = c * np_out + base_rows
        if has_feat:
            pltpu.sync_copy(args["facc"].at[pl.ds(base_rows, rows_per)],
                            args["facc_out"].at[pl.ds(obase, rows_per)])
        if has_sacc:
            pltpu.sync_copy(args["sacc"].at[pl.ds(base_rows, rows_per)],
                            args["sacc_out"].at[pl.ds(obase, rows_per)])

    mesh = plsc.VectorSubcoreMesh(**_MESH)
    return pl.kernel(body, out_type=tuple(out_type), mesh=mesh,
                     scratch_types=list(scratch.values()),
                     compiler_params=pltpu.CompilerParams(
                         needs_layout_passes=False,
                         use_tc_tiling_on_sc=False))


R = 2048
GN = 5


def _tc_call(body, out_type):
    return pl.pallas_call(body, out_shape=out_type)


def _rows(w):
    return pl.BlockSpec((R, w), lambda i: (i, 0))


def _acc3(w):
    return pl.BlockSpec((NC, R, w), lambda i: (0, i, 0))


def _full(*shape):
    nd = len(shape)
    return pl.BlockSpec(shape, lambda i: (0,) * nd)


def _vec():
    return pl.BlockSpec((R,), lambda i: (i,))


def _inv0(s3):
    s0 = s3[0, :, 0] + s3[1, :, 0]
    return jnp.where(s0 > 0, 1.0 / s0, 0.0)[:, None]


def _dis_body(sacc_ref, out_ref):
    a = sacc_ref[...]
    deg = a[0, :, 0] + a[1, :, 0]
    out_ref[...] = jnp.where(deg > 0, lax.rsqrt(jnp.maximum(deg, 1e-30)), 0.0)


def _hw_body(xin_ref, gp_ref, w_ref, b_ref, out_ref):
    gp = gp_ref[...]
    g = jax.nn.relu(gp[0] + gp[1])
    xin = xin_ref[...]
    gate = jax.nn.sigmoid(
        jnp.dot(xin, w_ref[...], preferred_element_type=jnp.float32)
        + b_ref[...])
    out_ref[...] = gate * g + (1.0 - gate) * xin


def _proj_body(x_ref, wh_ref, wt_ref, ah1_ref, ah2_ref, at1_ref, at2_ref,
               rah_ref, rat_ref,
               xrh_ref, xrt_ref, ph1_ref, ph2_ref, pt1_ref, pt2_ref,
               ehn_ref, etn_ref):
    x = x_ref[...]
    xrh = jnp.dot(x, wh_ref[...], preferred_element_type=jnp.float32)
    xrt = jnp.dot(x, wt_ref[...], preferred_element_type=jnp.float32)
    xrh_ref[...] = xrh
    xrt_ref[...] = xrt
    ph1_ref[...] = jnp.sum(xrh * ah1_ref[...], axis=1)
    ph2_ref[...] = jnp.sum(xrt * ah2_ref[...], axis=1)
    pt1_ref[...] = jnp.sum(xrh * at1_ref[...], axis=1)
    pt2_ref[...] = jnp.sum(xrt * at2_ref[...], axis=1)
    ehn_ref[...] = jnp.sum(x * rah_ref[...], axis=1)
    etn_ref[...] = jnp.sum(x * rat_ref[...], axis=1)


def _xr_body(fh_ref, sh_ref, ft_ref, st_ref, ar_ref, xr_ref, rp_ref):
    fh = fh_ref[...]
    ft = ft_ref[...]
    sh = sh_ref[...]
    st = st_ref[...]
    sh0 = sh[0, :NREL, 0] + sh[1, :NREL, 0]
    st0 = st[0, :NREL, 0] + st[1, :NREL, 0]
    inv_h = jnp.where(sh0 > 0, 1.0 / sh0, 0.0)[:, None]
    inv_t = jnp.where(st0 > 0, 1.0 / st0, 0.0)[:, None]
    xr = (fh[0, :NREL, :] + fh[1, :NREL, :]) * inv_h \
        + (ft[0, :NREL, :] + ft[1, :NREL, :]) * inv_t
    xr_ref[...] = xr
    rp_ref[...] = jnp.sum(xr * ar_ref[...], axis=1)


def _cat_body(x_ref, fh_ref, sh_ref, ft_ref, st_ref, ai_ref, aj_ref,
              xcat_ref, gi_ref, gj_ref):
    fh = fh_ref[...]
    ft = ft_ref[...]
    xeh = (fh[0] + fh[1]) * _inv0(sh_ref[...])
    xet = (ft[0] + ft[1]) * _inv0(st_ref[...])
    xcat = jnp.concatenate([x_ref[...], xeh, xet], axis=1)
    xcat_ref[...] = xcat
    gi_ref[...] = jnp.sum(xcat * ai_ref[...], axis=1)
    gj_ref[...] = jnp.sum(xcat * aj_ref[...], axis=1)


def _out_body(xcat_ref, fg_ref, sg_ref, out_ref):
    fg = fg_ref[...]
    xg = jax.nn.relu((fg[0] + fg[1]) * _inv0(sg_ref[...]))
    out_ref[...] = jnp.concatenate([xcat_ref[...], xg], axis=1)


def _padi(a, fill):
    return jnp.concatenate(
        [a, jnp.full((E_PAD - E,), fill, a.dtype)])


@jax.jit
def _run(x_e, edge_index, rel, edge_index_all,
         hw1_W, hw1_b, hw2_W, hw2_b,
         e2r_ah1, e2r_ah2, e2r_at1, e2r_at2, e2r_wh, e2r_wt,
         r2e_ah, r2e_at, r2e_ar, gat_ai, gat_aj):
    f32 = jnp.float32
    src_a = edge_index_all[0]
    dst_a = edge_index_all[1]
    h = edge_index[0]
    t = edge_index[1]

    src_a_g = _padi(src_a, 0)
    dst_a_g = _padi(dst_a, 0)
    dst_a_s = _padi(dst_a, N).reshape(E_PAD // K, K)
    h_g = _padi(h, 0)
    t_g = _padi(t, 0)
    rel_g = _padi(rel, 0)
    h_s = _padi(h, N).reshape(E_PAD // K, K)
    t_s = _padi(t, N).reshape(E_PAD // K, K)
    rel_s = _padi(rel, NREL).reshape(E_PAD // K, K)

    deg_pass = _sc_edge_pass("ones", 0, N, NP, 0, 0, False, True, False, 0)
    (sacc_deg,) = deg_pass(dst_a_s)
    dis = pl.pallas_call(
        _dis_body, grid=(GN,), in_specs=[_acc3(L)], out_specs=_vec(),
        out_shape=jax.ShapeDtypeStruct((N,), f32))(
        sacc_deg.reshape(NC, NP, L))

    gcn = _sc_edge_pass("gcn", EH, N, NP, N, N, True, False, False, N,
                        uv_same=True, i1_is_gidx=True)
    (g1,) = gcn(x_e, src_a_g, dst_a_s, dst_a_g, dis)
    hw_call = pl.pallas_call(
        _hw_body, grid=(GN,),
        in_specs=[_rows(EH), _acc3(EH), _full(EH, EH), _full(1, EH)],
        out_specs=_rows(EH),
        out_shape=jax.ShapeDtypeStruct((N, EH), f32))
    x1 = hw_call(x_e, g1.reshape(NC, NP, EH), hw1_W, hw1_b.reshape(1, EH))

    (g2,) = gcn(x1, src_a_g, dst_a_s, dst_a_g, dis)
    x = hw_call(x1, g2.reshape(NC, NP, EH), hw2_W, hw2_b.reshape(1, EH))

    outs = pl.pallas_call(
        _proj_body, grid=(GN,),
        in_specs=[_rows(EH), _full(EH, RH), _full(EH, RH)]
        + [_full(1, RH)] * 4 + [_full(1, EH)] * 2,
        out_specs=(_rows(RH), _rows(RH)) + (_vec(),) * 6,
        out_shape=(
            jax.ShapeDtypeStruct((N, RH), f32),
            jax.ShapeDtypeStruct((N, RH), f32),
        ) + (jax.ShapeDtypeStruct((N,), f32),) * 6,
    )(x, e2r_wh, e2r_wt,
      e2r_ah1.reshape(1, RH), e2r_ah2.reshape(1, RH),
      e2r_at1.reshape(1, RH), e2r_at2.reshape(1, RH),
      r2e_ah.reshape(1, EH), r2e_at.reshape(1, EH))
    xrh, xrt, ph1, ph2, pt1, pt2, ehn, etn = outs

    e2r = _sc_edge_pass("gat", RH, NREL, NRELP, N, N, True, True, False, N)
    fh, sh = e2r(xrh, h_g, rel_s, h_g, t_g, ph1, ph2)
    ft, st = e2r(xrt, t_g, rel_s, h_g, t_g, pt1, pt2)
    x_r, r_proj = _tc_call(_xr_body, (
        jax.ShapeDtypeStruct((NREL, RH), f32),
        jax.ShapeDtypeStruct((NREL,), f32),
    ))(fh.reshape(NC, NRELP, RH), sh.reshape(NC, NRELP, L),
       ft.reshape(NC, NRELP, RH), st.reshape(NC, NRELP, L),
       r2e_ar.reshape(1, RH))

    r2e = _sc_edge_pass("gat", RH, N, NP, N, NREL, True, True, False, NREL)
    fxh, sxh = r2e(x_r, rel_g, h_s, h_g, rel_g, ehn, r_proj)
    fxt, sxt = r2e(x_r, rel_g, t_s, t_g, rel_g, etn, r_proj)
    dcat = EH + 2 * RH
    xcat, gi, gj = pl.pallas_call(
        _cat_body, grid=(GN,),
        in_specs=[_rows(EH), _acc3(RH), _acc3(L), _acc3(RH), _acc3(L),
                  _full(1, dcat), _full(1, dcat)],
        out_specs=(_rows(dcat), _vec(), _vec()),
        out_shape=(
            jax.ShapeDtypeStruct((N, dcat), f32),
            jax.ShapeDtypeStruct((N,), f32),
            jax.ShapeDtypeStruct((N,), f32),
        ),
    )(x, fxh.reshape(NC, NP, RH), sxh.reshape(NC, NP, L),
      fxt.reshape(NC, NP, RH), sxt.reshape(NC, NP, L),
      gat_ai.reshape(1, dcat), gat_aj.reshape(1, dcat))

    fin_a = _sc_edge_pass("gat", 0, N, NP, N, N, False, True, True, 0)
    sg, w_all = fin_a(dst_a_s, dst_a_g, src_a_g, gi, gj)
    dh = dcat // 2
    fin_b = _sc_edge_pass("load", dh, N, NP, 0, 0, True, False, False, N)
    (fg0,) = fin_b(xcat[:, :dh], src_a_g, dst_a_s, w_all)
    (fg1,) = fin_b(xcat[:, dh:], src_a_g, dst_a_s, w_all)
    fg = jnp.concatenate([fg0.reshape(NC, NP, dh), fg1.reshape(NC, NP, dh)],
                         axis=2)

    return pl.pallas_call(
        _out_body, grid=(GN,),
        in_specs=[_rows(dcat), _acc3(dcat), _acc3(L)],
        out_specs=_rows(2 * dcat),
        out_shape=jax.ShapeDtypeStruct((N, 2 * dcat), f32))(
        xcat, fg, sg.reshape(NC, NP, L))


def kernel(x_e, edge_index, rel, edge_index_all, rel_all, hw1_W, hw1_b,
           hw2_W, hw2_b, e2r_ah1, e2r_ah2, e2r_at1, e2r_at2, e2r_wh,
           e2r_wt, r2e_ah, r2e_at, r2e_ar, gat_ai, gat_aj):
    return _run(x_e, edge_index, rel, edge_index_all,
                hw1_W, hw1_b, hw2_W, hw2_b,
                e2r_ah1, e2r_ah2, e2r_at1, e2r_at2, e2r_wh, e2r_wt,
                r2e_ah, r2e_at, r2e_ar, gat_ai, gat_aj)


# speedup vs baseline: 32.9075x; 1.0016x over previous
import functools

import jax
import jax.numpy as jnp
from jax import lax
from jax.experimental import pallas as pl
from jax.experimental.pallas import tpu as pltpu
from jax.experimental.pallas import tpu_sc as plsc

N = 10000
E = 640000
EH = 128
RH = 32
NREL = 1000

NC = 2
NS = 16
NW = NC * NS
L = 16

K = 128
NJ = 10
CHUNK = NJ * K
NCHUNK = 16
E_PER = NCHUNK * CHUNK
E_PAD = NW * E_PER

NP = 10240
NRELP = 1024

_MESH = dict(core_axis_name="c", subcore_axis_name="s", num_cores=NC,
             num_subcores=NS)


def _sc_edge_pass(mode, d, n_out, np_out, n_u, n_v, has_feat, has_sacc,
                  w_out, n_in, uv_same=False, i1_is_gidx=False):
    out_type = []
    if has_feat:
        out_type.append(jax.ShapeDtypeStruct((NC * np_out, d), jnp.float32))
    if has_sacc:
        out_type.append(jax.ShapeDtypeStruct((NC * np_out, L), jnp.float32))
    if w_out:
        out_type.append(jax.ShapeDtypeStruct((E_PAD,), jnp.float32))

    scratch = {}
    if has_feat:
        scratch["facc"] = pltpu.VMEM_SHARED((np_out, d), jnp.float32)
        scratch["gidx_v"] = pltpu.VMEM((CHUNK,), jnp.int32)
        scratch["rows_v"] = pltpu.VMEM((2, K, d), jnp.float32)
    if has_sacc:
        scratch["sacc"] = pltpu.VMEM_SHARED((np_out, L), jnp.float32)
        scratch["brows_v"] = pltpu.VMEM((K, L), jnp.float32)
    scratch["sidx_v"] = pltpu.VMEM((NJ, K), jnp.int32)
    if mode in ("gat", "gcn"):
        if not i1_is_gidx:
            scratch["i1_v"] = pltpu.VMEM((CHUNK,), jnp.int32)
        scratch["i2_v"] = pltpu.VMEM((CHUNK,), jnp.int32)
        scratch["u_v"] = pltpu.VMEM((n_u,), jnp.float32)
        if not uv_same:
            scratch["v_v"] = pltpu.VMEM((n_v,), jnp.float32)
    if mode != "ones":
        scratch["w_v"] = pltpu.VMEM((CHUNK,), jnp.float32)
    scratch["sem"] = pltpu.SemaphoreType.DMA
    if has_feat:
        scratch["sem2"] = pltpu.SemaphoreType.DMA
        scratch["ssem"] = pltpu.SemaphoreType.DMA
        scratch["ssem2"] = pltpu.SemaphoreType.DMA
    snames = list(scratch.keys())

    def body(*refs):
        nin = 0
        args = {}
        if has_feat:
            args["x_hbm"] = refs[nin]; nin += 1
            args["gidx_hbm"] = refs[nin]; nin += 1
        args["sidx_hbm"] = refs[nin]; nin += 1
        if mode in ("gat", "gcn"):
            if not i1_is_gidx:
                args["i1_hbm"] = refs[nin]; nin += 1
            args["i2_hbm"] = refs[nin]; nin += 1
            args["u_hbm"] = refs[nin]; nin += 1
            if not uv_same:
                args["v_hbm"] = refs[nin]; nin += 1
        if mode == "load":
            args["wsrc_hbm"] = refs[nin]; nin += 1
        if has_feat:
            args["facc_out"] = refs[nin]; nin += 1
        if has_sacc:
            args["sacc_out"] = refs[nin]; nin += 1
        if w_out:
            args["w_hbm"] = refs[nin]; nin += 1
        for nm, r in zip(snames, refs[nin:]):
            args[nm] = r
        if mode in ("gat", "gcn"):
            if i1_is_gidx:
                args["i1_v"] = args["gidx_v"]
            if uv_same:
                args["v_v"] = args["u_v"]

        c = lax.axis_index("c")
        s = lax.axis_index("s")
        wid = c * NS + s

        if mode in ("gat", "gcn"):
            pltpu.sync_copy(args["u_hbm"], args["u_v"])
            if not uv_same:
                pltpu.sync_copy(args["v_hbm"], args["v_v"])

        zero16 = jnp.zeros((L,), jnp.float32)
        rows_per = np_out // NS
        base_rows = s * rows_per

        def zero_buf(zref, width):
            def zrow(i, _):
                for dd in range(width // L):
                    zref[i, pl.ds(dd * L, L)] = zero16
                return 0
            lax.fori_loop(0, K, zrow, 0)

        def zero_acc(zref, acc):
            zr = min(rows_per, K)
            def zstep(r, _):
                pltpu.sync_copy(zref.at[pl.ds(0, zr)],
                                acc.at[pl.ds(base_rows + r * zr, zr)])
                return 0
            lax.fori_loop(0, rows_per // zr, zstep, 0)

        if has_feat:
            zero_buf(args["rows_v"].at[0], d)
            zero_acc(args["rows_v"].at[0], args["facc"])
        if has_sacc:
            zero_buf(args["brows_v"], L)
            zero_acc(args["brows_v"], args["sacc"])
        plsc.subcore_barrier()

        if mode == "ones":
            one16 = jnp.full((L,), 1.0, jnp.float32)

            def orow(i, _):
                args["brows_v"][i, :] = one16
                return 0
            lax.fori_loop(0, K, orow, 0)

        sems = [args.get("sem"), args.get("sem2")]

        def chunk_body(ci, _):
            ebase = (wid * NCHUNK + ci) * CHUNK
            rbase = (wid * NCHUNK + ci) * NJ
            if has_feat:
                pltpu.sync_copy(args["gidx_hbm"].at[pl.ds(ebase, CHUNK)],
                                args["gidx_v"])

            def gather(j, b):
                return pltpu.async_copy(
                    args["x_hbm"].at[args["gidx_v"].at[pl.ds(j * K, K)]],
                    args["rows_v"].at[b], sems[b])

            cps = {}
            if has_feat:
                cps[0] = gather(0, 0)
            pltpu.sync_copy(args["sidx_hbm"].at[pl.ds(rbase, NJ)],
                            args["sidx_v"])
            if mode in ("gat", "gcn"):
                if not i1_is_gidx:
                    pltpu.sync_copy(args["i1_hbm"].at[pl.ds(ebase, CHUNK)],
                                    args["i1_v"])
                pltpu.sync_copy(args["i2_hbm"].at[pl.ds(ebase, CHUNK)],
                                args["i2_v"])

                def wstep(i, _):
                    off = i * L
                    idx1 = args["i1_v"][pl.ds(off, L)]
                    idx2 = args["i2_v"][pl.ds(off, L)]
                    a = plsc.load_gather(args["u_v"], [idx1])
                    b = plsc.load_gather(args["v_v"], [idx2])
                    if mode == "gat":
                        z = a + b
                        w16 = jnp.exp(jnp.maximum(z, 0.01 * z))
                    else:
                        w16 = a * b
                    args["w_v"][pl.ds(off, L)] = w16
                    return 0
                lax.fori_loop(0, CHUNK // L, wstep, 0)
            if mode == "load":
                pltpu.sync_copy(args["wsrc_hbm"].at[pl.ds(ebase, CHUNK)],
                                args["w_v"])
            if w_out:
                pltpu.sync_copy(args["w_v"],
                                args["w_hbm"].at[pl.ds(ebase, CHUNK)])

            ssems = [args.get("ssem"), args.get("ssem2")]
            scat = {}
            for j in range(NJ):
                b = j & 1
                if has_feat:
                    cps[b].wait()
                    if j + 1 < NJ:
                        if (1 - b) in scat:
                            scat[1 - b].wait()
                        cps[1 - b] = gather(j + 1, 1 - b)
                    rbuf = args["rows_v"].at[b]
                if mode != "ones":
                    def scale_grp(ii, _):
                        gbase = ii * L
                        w16 = args["w_v"][pl.ds(j * K + gbase, L)]
                        for l in range(L):
                            wb = lax.broadcast(w16[l], (L,))
                            row = gbase + l
                            if has_feat:
                                vals = [rbuf[row, pl.ds(dd * L, L)]
                                        for dd in range(d // L)]
                                for dd in range(d // L):
                                    rbuf[row, pl.ds(dd * L, L)] = (
                                        vals[dd] * wb)
                            if has_sacc:
                                args["brows_v"][row, :] = wb
                        return 0
                    lax.fori_loop(0, K // L, scale_grp, 0)
                row_idx = args["sidx_v"].at[j]
                if has_feat:
                    scat[b] = pltpu.async_copy(
                        rbuf, args["facc"].at[row_idx], ssems[b], add=True)
                if has_sacc:
                    pltpu.sync_copy(args["brows_v"],
                                    args["sacc"].at[row_idx], add=True)
            for b in list(scat):
                scat[b].wait()
            return 0
        lax.fori_loop(0, NCHUNK, chunk_body, 0)

        plsc.subcore_barrier()
        obase = c * np_out + base_rows
        if has_feat:
            pltpu.sync_copy(args["facc"].at[pl.ds(base_rows, rows_per)],
                            args["facc_out"].at[pl.ds(obase, rows_per)])
        if has_sacc:
            pltpu.sync_copy(args["sacc"].at[pl.ds(base_rows, rows_per)],
                            args["sacc_out"].at[pl.ds(obase, rows_per)])

    mesh = plsc.VectorSubcoreMesh(**_MESH)
    return pl.kernel(body, out_type=tuple(out_type), mesh=mesh,
                     scratch_types=list(scratch.values()),
                     compiler_params=pltpu.CompilerParams(
                         needs_layout_passes=False,
                         use_tc_tiling_on_sc=False))


R = 2048
GN = 5


def _tc_call(body, out_type):
    return pl.pallas_call(body, out_shape=out_type)


def _rows(w):
    return pl.BlockSpec((R, w), lambda i: (i, 0))


def _acc3(w):
    return pl.BlockSpec((NC, R, w), lambda i: (0, i, 0))


def _full(*shape):
    nd = len(shape)
    return pl.BlockSpec(shape, lambda i: (0,) * nd)


def _vec():
    return pl.BlockSpec((R,), lambda i: (i,))


def _inv0(s3):
    s0 = s3[0, :, 0] + s3[1, :, 0]
    return jnp.where(s0 > 0, 1.0 / s0, 0.0)[:, None]


def _dis_body(sacc_ref, out_ref):
    a = sacc_ref[...]
    deg = a[0, :, 0] + a[1, :, 0]
    out_ref[...] = jnp.where(deg > 0, lax.rsqrt(jnp.maximum(deg, 1e-30)), 0.0)


def _hw_body(xin_ref, gp_ref, w_ref, b_ref, out_ref):
    gp = gp_ref[...]
    g = jax.nn.relu(gp[0] + gp[1])
    xin = xin_ref[...]
    gate = jax.nn.sigmoid(
        jnp.dot(xin, w_ref[...], preferred_element_type=jnp.float32)
        + b_ref[...])
    out_ref[...] = gate * g + (1.0 - gate) * xin


def _proj_body(x_ref, wh_ref, wt_ref, ah1_ref, ah2_ref, at1_ref, at2_ref,
               rah_ref, rat_ref,
               xrh_ref, xrt_ref, ph1_ref, ph2_ref, pt1_ref, pt2_ref,
               ehn_ref, etn_ref):
    x = x_ref[...]
    xrh = jnp.dot(x, wh_ref[...], preferred_element_type=jnp.float32)
    xrt = jnp.dot(x, wt_ref[...], preferred_element_type=jnp.float32)
    xrh_ref[...] = xrh
    xrt_ref[...] = xrt
    ph1_ref[...] = jnp.sum(xrh * ah1_ref[...], axis=1)
    ph2_ref[...] = jnp.sum(xrt * ah2_ref[...], axis=1)
    pt1_ref[...] = jnp.sum(xrh * at1_ref[...], axis=1)
    pt2_ref[...] = jnp.sum(xrt * at2_ref[...], axis=1)
    ehn_ref[...] = jnp.sum(x * rah_ref[...], axis=1)
    etn_ref[...] = jnp.sum(x * rat_ref[...], axis=1)


def _xr_body(fh_ref, sh_ref, ft_ref, st_ref, ar_ref, xr_ref, rp_ref):
    fh = fh_ref[...]
    ft = ft_ref[...]
    sh = sh_ref[...]
    st = st_ref[...]
    sh0 = sh[0, :NREL, 0] + sh[1, :NREL, 0]
    st0 = st[0, :NREL, 0] + st[1, :NREL, 0]
    inv_h = jnp.where(sh0 > 0, 1.0 / sh0, 0.0)[:, None]
    inv_t = jnp.where(st0 > 0, 1.0 / st0, 0.0)[:, None]
    xr = (fh[0, :NREL, :] + fh[1, :NREL, :]) * inv_h \
        + (ft[0, :NREL, :] + ft[1, :NREL, :]) * inv_t
    xr_ref[...] = xr
    rp_ref[...] = jnp.sum(xr * ar_ref[...], axis=1)


def _cat_body(x_ref, fh_ref, sh_ref, ft_ref, st_ref, ai_ref, aj_ref,
              xcat_ref, gi_ref, gj_ref):
    fh = fh_ref[...]
    ft = ft_ref[...]
    xeh = (fh[0] + fh[1]) * _inv0(sh_ref[...])
    xet = (ft[0] + ft[1]) * _inv0(st_ref[...])
    xcat = jnp.concatenate([x_ref[...], xeh, xet], axis=1)
    xcat_ref[...] = xcat
    gi_ref[...] = jnp.sum(xcat * ai_ref[...], axis=1)
    gj_ref[...] = jnp.sum(xcat * aj_ref[...], axis=1)


def _out_body(xcat_ref, fg_ref, sg_ref, out_ref):
    fg = fg_ref[...]
    xg = jax.nn.relu((fg[0] + fg[1]) * _inv0(sg_ref[...]))
    out_ref[...] = jnp.concatenate([xcat_ref[...], xg], axis=1)


def _padi(a, fill):
    return jnp.concatenate(
        [a, jnp.full((E_PAD - E,), fill, a.dtype)])


@jax.jit
def _run(x_e, edge_index, rel, edge_index_all,
         hw1_W, hw1_b, hw2_W, hw2_b,
         e2r_ah1, e2r_ah2, e2r_at1, e2r_at2, e2r_wh, e2r_wt,
         r2e_ah, r2e_at, r2e_ar, gat_ai, gat_aj):
    f32 = jnp.float32
    src_a = edge_index_all[0]
    dst_a = edge_index_all[1]
    h = edge_index[0]
    t = edge_index[1]

    src_a_g = _padi(src_a, 0)
    dst_a_g = _padi(dst_a, 0)
    dst_a_s = _padi(dst_a, N).reshape(E_PAD // K, K)
    h_g = _padi(h, 0)
    t_g = _padi(t, 0)
    rel_g = _padi(rel, 0)
    h_s = _padi(h, N).reshape(E_PAD // K, K)
    t_s = _padi(t, N).reshape(E_PAD // K, K)
    rel_s = _padi(rel, NREL).reshape(E_PAD // K, K)

    deg_pass = _sc_edge_pass("ones", 0, N, NP, 0, 0, False, True, False, 0)
    (sacc_deg,) = deg_pass(dst_a_s)
    dis = pl.pallas_call(
        _dis_body, grid=(GN,), in_specs=[_acc3(L)], out_specs=_vec(),
        out_shape=jax.ShapeDtypeStruct((N,), f32))(
        sacc_deg.reshape(NC, NP, L))

    gcn = _sc_edge_pass("gcn", EH, N, NP, N, N, True, False, False, N,
                        uv_same=True, i1_is_gidx=True)
    (g1,) = gcn(x_e, src_a_g, dst_a_s, dst_a_g, dis)
    hw_call = pl.pallas_call(
        _hw_body, grid=(GN,),
        in_specs=[_rows(EH), _acc3(EH), _full(EH, EH), _full(1, EH)],
        out_specs=_rows(EH),
        out_shape=jax.ShapeDtypeStruct((N, EH), f32))
    x1 = hw_call(x_e, g1.reshape(NC, NP, EH), hw1_W, hw1_b.reshape(1, EH))

    (g2,) = gcn(x1, src_a_g, dst_a_s, dst_a_g, dis)
    x = hw_call(x1, g2.reshape(NC, NP, EH), hw2_W, hw2_b.reshape(1, EH))

    outs = pl.pallas_call(
        _proj_body, grid=(GN,),
        in_specs=[_rows(EH), _full(EH, RH), _full(EH, RH)]
        + [_full(1, RH)] * 4 + [_full(1, EH)] * 2,
        out_specs=(_rows(RH), _rows(RH)) + (_vec(),) * 6,
        out_shape=(
            jax.ShapeDtypeStruct((N, RH), f32),
            jax.ShapeDtypeStruct((N, RH), f32),
        ) + (jax.ShapeDtypeStruct((N,), f32),) * 6,
    )(x, e2r_wh, e2r_wt,
      e2r_ah1.reshape(1, RH), e2r_ah2.reshape(1, RH),
      e2r_at1.reshape(1, RH), e2r_at2.reshape(1, RH),
      r2e_ah.reshape(1, EH), r2e_at.reshape(1, EH))
    xrh, xrt, ph1, ph2, pt1, pt2, ehn, etn = outs

    e2r = _sc_edge_pass("gat", RH, NREL, NRELP, N, N, True, True, False, N)
    fh, sh = e2r(xrh, h_g, rel_s, h_g, t_g, ph1, ph2)
    ft, st = e2r(xrt, t_g, rel_s, h_g, t_g, pt1, pt2)
    x_r, r_proj = _tc_call(_xr_body, (
        jax.ShapeDtypeStruct((NREL, RH), f32),
        jax.ShapeDtypeStruct((NREL,), f32),
    ))(fh.reshape(NC, NRELP, RH), sh.reshape(NC, NRELP, L),
       ft.reshape(NC, NRELP, RH), st.reshape(NC, NRELP, L),
       r2e_ar.reshape(1, RH))

    r2e = _sc_edge_pass("gat", RH, N, NP, N, NREL, True, True, False, NREL)
    fxh, sxh = r2e(x_r, rel_g, h_s, h_g, rel_g, ehn, r_proj)
    fxt, sxt = r2e(x_r, rel_g, t_s, t_g, rel_g, etn, r_proj)
    dcat = EH + 2 * RH
    xcat, gi, gj = pl.pallas_call(
        _cat_body, grid=(GN,),
        in_specs=[_rows(EH), _acc3(RH), _acc3(L), _acc3(RH), _acc3(L),
                  _full(1, dcat), _full(1, dcat)],
        out_specs=(_rows(dcat), _vec(), _vec()),
        out_shape=(
            jax.ShapeDtypeStruct((N, dcat), f32),
            jax.ShapeDtypeStruct((N,), f32),
            jax.ShapeDtypeStruct((N,), f32),
        ),
    )(x, fxh.reshape(NC, NP, RH), sxh.reshape(NC, NP, L),
      fxt.reshape(NC, NP, RH), sxt.reshape(NC, NP, L),
      gat_ai.reshape(1, dcat), gat_aj.reshape(1, dcat))

    fin_a = _sc_edge_pass("gat", 0, N, NP, N, N, False, True, True, 0)
    sg, w_all = fin_a(dst_a_s, dst_a_g, src_a_g, gi, gj)
    dh = dcat // 2
    fin_b = _sc_edge_pass("load", dh, N, NP, 0, 0, True, False, False, N)
    (fg0,) = fin_b(xcat[:, :dh], src_a_g, dst_a_s, w_all)
    (fg1,) = fin_b(xcat[:, dh:], src_a_g, dst_a_s, w_all)
    fg = jnp.concatenate([fg0.reshape(NC, NP, dh), fg1.reshape(NC, NP, dh)],
                         axis=2)

    return pl.pallas_call(
        _out_body, grid=(GN,),
        in_specs=[_rows(dcat), _acc3(dcat), _acc3(L)],
        out_specs=_rows(2 * dcat),
        out_shape=jax.ShapeDtypeStruct((N, 2 * dcat), f32))(
        xcat, fg, sg.reshape(NC, NP, L))


def kernel(x_e, edge_index, rel, edge_index_all, rel_all, hw1_W, hw1_b,
           hw2_W, hw2_b, e2r_ah1, e2r_ah2, e2r_at1, e2r_at2, e2r_wh,
           e2r_wt, r2e_ah, r2e_at, r2e_ar, gat_ai, gat_aj):
    return _run(x_e, edge_index, rel, edge_index_all,
                hw1_W, hw1_b, hw2_W, hw2_b,
                e2r_ah1, e2r_ah2, e2r_at1, e2r_at2, e2r_wh, e2r_wt,
                r2e_ah, r2e_at, r2e_ar, gat_ai, gat_aj)
